# baseline (speedup 1.0000x reference)
import functools

import jax
import jax.numpy as jnp
from jax import lax
from jax.experimental import pallas as pl
from jax.experimental.pallas import tpu as pltpu
from jax.experimental.pallas import tpu_sc as plsc

SLOTS = 65536
D = 256

S_SC = 8192
S_TC = SLOTS - S_SC

B = 4096
NJ = S_TC // B

NC, NS, L = 2, 16, 16
NW = NC * NS
RW = S_SC // NW
NT = D // L

_DN = (((1,), (0,)), ((), ()))

_sc_mesh = plsc.VectorSubcoreMesh(core_axis_name="c", subcore_axis_name="s")


def _bf16r(x):
    return x.astype(jnp.bfloat16).astype(jnp.float32)


def _bf16r_sc(x):
    t = x * 65537.0
    return t - (t - x)


@functools.partial(
    pl.kernel, mesh=_sc_mesh,
    out_type=jax.ShapeDtypeStruct((NW, RW // L, L), jnp.float32),
    scratch_types=[
        pltpu.VMEM((D,), jnp.float32),
        pltpu.VMEM((RW, D), jnp.float32),
        pltpu.VMEM((RW // L, L), jnp.float32),
        pltpu.VMEM((2 * L,), jnp.float32),
    ],
)
def _sc_sims(c_hbm, keys_hbm, sims_hbm, cbuf, kbuf, simsb, tb):
    wid = lax.axis_index("s") * NC + lax.axis_index("c")
    base = wid * RW
    pltpu.sync_copy(c_hbm, cbuf)
    pltpu.sync_copy(keys_hbm.at[pl.ds(base, RW)], kbuf)
    lanes = lax.iota(jnp.int32, L)

    def block(g, carry):
        sv = jnp.zeros((L,), jnp.float32)
        for r in range(L):
            rr = g * L + r
            acc = jnp.zeros((L,), jnp.float32)
            for t in range(NT):
                kv = _bf16r_sc(kbuf[rr, pl.ds(t * L, L)])
                acc = acc + kv * cbuf[pl.ds(t * L, L)]
            tot = acc
            for sh in (8, 4, 2, 1):
                tb[pl.ds(0, L)] = tot
                tb[pl.ds(L, L)] = tot
                tot = tot + tb[pl.ds(sh, L)]
            sv = jnp.where(lanes == r, tot, sv)
        simsb[g, :] = sv
        return carry

    lax.fori_loop(0, RW // L, block, 0)
    pltpu.sync_copy(simsb, sims_hbm.at[wid])


@functools.partial(
    pl.kernel, mesh=_sc_mesh,
    out_type=jax.ShapeDtypeStruct((NC, NT, L), jnp.float32),
    scratch_types=[
        pltpu.VMEM((RW,), jnp.float32),
        pltpu.VMEM((RW, D), jnp.float32),
        pltpu.VMEM((2 * L,), jnp.float32),
        pltpu.VMEM((NT, L), jnp.float32),
        pltpu.VMEM((NT, L), jnp.float32),
        pltpu.VMEM_SHARED((NT, L), jnp.float32),
    ],
)
def _sc_wsum(alpha_hbm, vals_hbm, out_hbm, abuf, vbuf, tb, vaccb, zbuf, shacc):
    cid = lax.axis_index("c")
    sid = lax.axis_index("s")
    wid = sid * NC + cid
    base = wid * RW
    pltpu.sync_copy(alpha_hbm.at[pl.ds(base, RW)], abuf)
    pltpu.sync_copy(vals_hbm.at[pl.ds(base, RW)], vbuf)
    lanes = lax.iota(jnp.int32, L)

    accs0 = tuple(jnp.zeros((L,), jnp.float32) for _ in range(NT))

    def block(g, accs):
        a16 = abuf[pl.ds(g * L, L)]
        for r in range(L):
            rr = g * L + r
            av = jnp.where(lanes == r, a16, 0.0)
            for sh in (8, 4, 2, 1):
                tb[pl.ds(0, L)] = av
                tb[pl.ds(L, L)] = av
                av = av + tb[pl.ds(sh, L)]
            av = _bf16r_sc(av)
            accs = tuple(
                accs[t] + av * _bf16r_sc(vbuf[rr, pl.ds(t * L, L)])
                for t in range(NT)
            )
        return accs

    accs = lax.fori_loop(0, RW // L, block, accs0)
    for t in range(NT):
        vaccb[t, :] = accs[t]
        zbuf[t, :] = jnp.zeros((L,), jnp.float32)

    @pl.when(sid == 0)
    def _init():
        pltpu.sync_copy(zbuf, shacc)

    plsc.subcore_barrier()
    pltpu.sync_copy(vaccb, shacc.at[lax.iota(jnp.int32, NT)], add=True)
    plsc.subcore_barrier()

    @pl.when(sid == 0)
    def _emit():
        pltpu.sync_copy(shacc, out_hbm.at[cid])


def _tc_sims(c_ref, keys_ref, sims_ref, m_ref, m_s):
    j = pl.program_id(0)
    sims = jax.lax.dot_general(keys_ref[...], c_ref[...], _DN,
                               preferred_element_type=jnp.float32)
    sims_ref[...] = sims
    bmax = jnp.max(sims)
    prev = jnp.where(j == 0, -jnp.inf, m_s[0])
    m_s[0] = jnp.maximum(prev, bmax)

    @pl.when(j == NJ - 1)
    def _emit():
        m_ref[...] = m_s[0].reshape(1, 1)


def _tc_softmax(sims_tc_ref, m_ref, sims_sc_ref, a_tc_ref, a_sc_ref):
    m = jnp.maximum(m_ref[0, 0], jnp.max(sims_sc_ref[...]))
    e_tc = jnp.exp(sims_tc_ref[...] - m)
    e_sc = jnp.exp(sims_sc_ref[...] - m)
    inv = 1.0 / (jnp.sum(e_tc) + jnp.sum(e_sc))
    a_tc_ref[...] = e_tc * inv
    a_sc_ref[...] = e_sc * inv


def _tc_wsum(a_ref, vals_ref, vhat_ref, acc_v):
    j = pl.program_id(0)
    ab = a_ref[pl.ds(j, 1)].reshape(1, B)
    part = jax.lax.dot_general(ab, vals_ref[...], _DN,
                               preferred_element_type=jnp.float32)
    prev = jnp.where(j == 0, jnp.zeros((1, D), jnp.float32), acc_v[...])
    acc_v[...] = prev + part

    @pl.when(j == NJ - 1)
    def _emit():
        vhat_ref[...] = acc_v[...]


@jax.jit
def kernel(c, keys, vals):
    c2 = c.reshape(D, 1)
    c_sc = _bf16r(c)
    keys_tc, keys_sc = keys[:S_TC], keys[S_TC:]
    vals_tc, vals_sc = vals[:S_TC], vals[S_TC:]

    sims_sc = _sc_sims(c_sc, keys_sc).reshape(S_SC)

    sims_tc, m = pl.pallas_call(
        _tc_sims,
        grid=(NJ,),
        in_specs=[
            pl.BlockSpec((D, 1), lambda j: (0, 0)),
            pl.BlockSpec((B, D), lambda j: (j, 0)),
        ],
        out_specs=[
            pl.BlockSpec((B, 1), lambda j: (j, 0)),
            pl.BlockSpec((1, 1), lambda j: (0, 0)),
        ],
        out_shape=[
            jax.ShapeDtypeStruct((S_TC, 1), jnp.float32),
            jax.ShapeDtypeStruct((1, 1), jnp.float32),
        ],
        scratch_shapes=[pltpu.SMEM((1,), jnp.float32)],
        compiler_params=pltpu.CompilerParams(
            dimension_semantics=("arbitrary",),
        ),
    )(c2, keys_tc)

    a_tc3, a_sc2 = pl.pallas_call(
        _tc_softmax,
        grid=(1,),
        in_specs=[
            pl.BlockSpec((NJ, 1, B), lambda i: (0, 0, 0)),
            pl.BlockSpec((1, 1), lambda i: (0, 0)),
            pl.BlockSpec((1, S_SC), lambda i: (0, 0)),
        ],
        out_specs=[
            pl.BlockSpec((NJ, 1, B), lambda i: (0, 0, 0)),
            pl.BlockSpec((1, S_SC), lambda i: (0, 0)),
        ],
        out_shape=[
            jax.ShapeDtypeStruct((NJ, 1, B), jnp.float32),
            jax.ShapeDtypeStruct((1, S_SC), jnp.float32),
        ],
    )(sims_tc.reshape(NJ, 1, B), m, sims_sc.reshape(1, S_SC))

    vhat_sc3 = _sc_wsum(a_sc2.reshape(S_SC), vals_sc)
    vhat_sc2 = vhat_sc3.reshape(NC, D)

    vhat_tc, = pl.pallas_call(
        _tc_wsum,
        grid=(NJ,),
        in_specs=[
            pl.BlockSpec((NJ, 1, B), lambda j: (0, 0, 0)),
            pl.BlockSpec((B, D), lambda j: (j, 0)),
        ],
        out_specs=[pl.BlockSpec((1, D), lambda j: (0, 0))],
        out_shape=[jax.ShapeDtypeStruct((1, D), jnp.float32)],
        scratch_shapes=[pltpu.VMEM((1, D), jnp.float32)],
        compiler_params=pltpu.CompilerParams(
            dimension_semantics=("arbitrary",),
        ),
    )(a_tc3, vals_tc)

    vhat = vhat_tc.reshape(D) + vhat_sc2[0] + vhat_sc2[1]
    alpha = jnp.concatenate([a_tc3.reshape(S_TC), a_sc2.reshape(S_SC)])
    return (vhat, alpha)
